# Initial kernel scaffold; baseline (speedup 1.0000x reference)
import jax
import jax.numpy as jnp
from jax.experimental import pallas as pl


def kernel(alpha, beta, theta):
    raise NotImplementedError("write your pallas kernel here")


# speedup vs baseline: 42.5719x; 42.5719x over previous
import numpy as np
import jax
import jax.numpy as jnp
from jax.experimental import pallas as pl

_SW = 8
_N_NODE = 64
_N_STAGES = 4
_N_OPS = 8192
_N_ROWS = _N_STAGES * _N_NODE * _SW


def _pairs():
    I, J = [], []
    for i in range(2, _N_NODE + 1):
        for j in range(i + 1, _N_NODE + 2):
            I.append(i)
            J.append(j)
    return (np.asarray(I, np.int32).reshape(-1, 1),
            np.asarray(J, np.int32).reshape(-1, 1))


_I_IDX, _J_IDX = _pairs()


def _rowmax_body(a_ref, o_ref):
    x = a_ref[...]
    lane = jax.lax.broadcasted_iota(jnp.int32, x.shape, 1)
    x = jnp.where(lane < _N_OPS - 1, x, -jnp.inf)
    o_ref[...] = jnp.max(x, axis=1, keepdims=True)


def _rowmax(a2):
    return pl.pallas_call(
        _rowmax_body,
        grid=(16,),
        in_specs=[pl.BlockSpec((128, _N_OPS), lambda i: (i, 0))],
        out_specs=pl.BlockSpec((128, 1), lambda i: (i, 0)),
        out_shape=jax.ShapeDtypeStruct((_N_ROWS, 1), jnp.float32),
    )(a2)


def _epilogue_body(em_ref, beta_ref, i_ref, j_ref, theta_ref, out_ref):
    em = em_ref[...]
    m = jnp.max(em, axis=1, keepdims=True)
    p = jnp.exp(em - m)
    e = p / jnp.sum(p, axis=1, keepdims=True)

    ED = jnp.zeros((_N_STAGES, 128), jnp.float32)
    lane = jax.lax.broadcasted_iota(jnp.int32, (_N_STAGES, 128), 1)
    for j in range(2, _N_NODE + 2):
        rows = jnp.concatenate(
            [e[s * _N_NODE + j - 2][None, :] for s in range(_N_STAGES)], axis=0
        )
        if j < _SW:
            contrib = jnp.sum(rows[:, :j] * (ED[:, :j] + 1.0), axis=1,
                              keepdims=True)
        else:
            contrib = jnp.sum(rows * (ED[:, j - _SW:j] + 1.0), axis=1,
                              keepdims=True)
        ED = jnp.where(lane == j, ED + contrib, ED)

    beta = beta_ref[...]
    bm = jnp.max(beta, axis=1, keepdims=True)
    be = jnp.exp(beta - bm)
    denom = jnp.sum(be, axis=1, keepdims=True)
    n_iota = jax.lax.broadcasted_iota(jnp.int32, (_I_IDX.shape[0], 128), 1)
    oh = ((i_ref[...] == n_iota).astype(jnp.float32) +
          (j_ref[...] == n_iota).astype(jnp.float32))
    T = jax.lax.dot_general(be, oh, (((1,), (0,)), ((), ())),
                            preferred_element_type=jnp.float32)
    depth = jnp.sum(T * ED, axis=1, keepdims=True) / denom
    out_ref[...] = jnp.sum(theta_ref[...] * depth, axis=0, keepdims=True)


def _epilogue(em, beta, theta):
    return pl.pallas_call(
        _epilogue_body,
        out_shape=jax.ShapeDtypeStruct((1, 1), jnp.float32),
    )(em, beta, jnp.asarray(_I_IDX), jnp.asarray(_J_IDX),
      theta.reshape(_N_STAGES, 1))


def kernel(alpha, beta, theta):
    a2 = alpha.reshape(_N_ROWS, _N_OPS)
    em = _rowmax(a2).reshape(_N_STAGES * _N_NODE, _SW)
    return _epilogue(em, beta, theta)[0, 0]
